# baseline (speedup 1.0000x reference)
import functools

import jax
import jax.numpy as jnp
import numpy as np
from jax import lax
from jax.experimental import pallas as pl
from jax.experimental.pallas import tpu as pltpu
from jax.experimental.pallas import tpu_sc as plsc


def _pack(y):
    d = y.shape[1]
    u = lax.bitcast_convert_type(
        y.astype(jnp.bfloat16), jnp.uint16).astype(jnp.int32)
    return u[:, :d // 2] | (u[:, d // 2:] << 16)


def _node_transform_body(x0_ref, x1_ref, w0_ref, w1_ref, b0_ref, b1_ref,
                         y0_ref, y1_ref):
    y0_ref[...] = _pack(
        jnp.dot(x0_ref[...], w0_ref[...], preferred_element_type=jnp.float32)
        + b0_ref[...])
    y1_ref[...] = _pack(
        jnp.dot(x1_ref[...], w1_ref[...], preferred_element_type=jnp.float32)
        + b1_ref[...])


def _node_transform(x0, x1, W0, b0, W1, b1):
    n, d = x0.shape
    grid = 10
    rows = n // grid
    row_spec = pl.BlockSpec((rows, d), lambda i: (i, 0))
    full_spec = pl.BlockSpec((d, d), lambda i: (0, 0))
    bias_spec = pl.BlockSpec((1, d), lambda i: (0, 0))
    packed_spec = pl.BlockSpec((rows, d // 2), lambda i: (i, 0))
    return pl.pallas_call(
        _node_transform_body,
        grid=(grid,),
        in_specs=[row_spec, row_spec, full_spec, full_spec, bias_spec,
                  bias_spec],
        out_specs=[packed_spec, packed_spec],
        out_shape=[
            jax.ShapeDtypeStruct((n, d // 2), jnp.int32),
            jax.ShapeDtypeStruct((n, d // 2), jnp.int32),
        ],
    )(x0, x1, W0, W1, b0.reshape(1, d), b1.reshape(1, d))


_CHUNK = 128
_NSLOT = 3


def _make_gather_add(E, D):
    info = plsc.get_sparse_core_info()
    nw = info.num_cores * info.num_subcores
    e_per_w = E // nw
    assert e_per_w * nw == E and e_per_w % 8 == 0
    n_full = e_per_w // _CHUNK
    tail = e_per_w - n_full * _CHUNK
    assert tail % 8 == 0
    dw = D // 2
    n_steps = n_full + 2
    n_outer = (n_steps + _NSLOT - 1) // _NSLOT
    mesh = plsc.VectorSubcoreMesh(core_axis_name="c", subcore_axis_name="s")

    @functools.partial(
        pl.kernel,
        mesh=mesh,
        out_type=jax.ShapeDtypeStruct((E, D), jnp.float32),
        compiler_params=pltpu.CompilerParams(use_tc_tiling_on_sc=False),
        scratch_types=(
            [pltpu.VMEM((e_per_w,), jnp.int32)] * 2
            + [pltpu.VMEM((_CHUNK, dw), jnp.int32)] * (2 * _NSLOT)
            + [pltpu.VMEM((_CHUNK, D), jnp.float32)] * _NSLOT
            + [pltpu.SemaphoreType.DMA] * (3 * _NSLOT)
        ),
    )
    def gather_add(y0_hbm, y1_hbm, src0_hbm, src1_hbm, out_hbm,
                   i0_all, i1_all, *bufs_and_sems):
        rbuf0 = bufs_and_sems[:_NSLOT]
        rbuf1 = bufs_and_sems[_NSLOT:2 * _NSLOT]
        wbuf = bufs_and_sems[2 * _NSLOT:3 * _NSLOT]
        sem_g0 = bufs_and_sems[3 * _NSLOT:4 * _NSLOT]
        sem_g1 = bufs_and_sems[4 * _NSLOT:5 * _NSLOT]
        sem_w = bufs_and_sems[5 * _NSLOT:6 * _NSLOT]
        wid = lax.axis_index("s") * info.num_cores + lax.axis_index("c")
        base = wid * e_per_w

        pltpu.sync_copy(src0_hbm.at[pl.ds(base, e_per_w)], i0_all)
        pltpu.sync_copy(src1_hbm.at[pl.ds(base, e_per_w)], i1_all)

        def idx0(c):
            return i0_all.at[pl.ds(c * _CHUNK, _CHUNK)]

        def idx1(c):
            return i1_all.at[pl.ds(c * _CHUNK, _CHUNK)]

        def out_slice(c):
            return out_hbm.at[pl.ds(base + c * _CHUNK, _CHUNK)]

        def expand_add(rb0, rb1, wb, count):
            mask = jnp.int32(-65536)

            @plsc.parallel_loop(0, count, unroll=16)
            def _(r):
                for q in range(D // 32):
                    w0 = rb0[r, pl.ds(16 * q, 16)]
                    w1 = rb1[r, pl.ds(16 * q, 16)]
                    lo = (lax.bitcast_convert_type(w0 << 16, jnp.float32)
                          + lax.bitcast_convert_type(w1 << 16, jnp.float32))
                    hi = (lax.bitcast_convert_type(w0 & mask, jnp.float32)
                          + lax.bitcast_convert_type(w1 & mask, jnp.float32))
                    wb[r, pl.ds(32 * q, 16)] = lo
                    wb[r, pl.ds(32 * q + 16, 16)] = hi

        def step(outer, k):
            c2 = outer * _NSLOT + k
            cA = c2
            cC = c2 - 2
            bA = k
            bC = (k - 2) % _NSLOT

            @pl.when(jnp.logical_and(cC >= 0, cC < n_full))
            def _():
                pltpu.make_async_copy(
                    y0_hbm.at[idx0(cC)], rbuf0[bC], sem_g0[bC]).wait()
                pltpu.make_async_copy(
                    y1_hbm.at[idx1(cC)], rbuf1[bC], sem_g1[bC]).wait()

                @pl.when(cC >= _NSLOT)
                def _():
                    pltpu.make_async_copy(
                        wbuf[bC], out_slice(cC - _NSLOT), sem_w[bC]).wait()

                expand_add(rbuf0[bC], rbuf1[bC], wbuf[bC], _CHUNK)
                pltpu.async_copy(wbuf[bC], out_slice(cC), sem_w[bC])

            @pl.when(cA < n_full)
            def _():
                pltpu.async_copy(y0_hbm.at[idx0(cA)], rbuf0[bA], sem_g0[bA])
                pltpu.async_copy(y1_hbm.at[idx1(cA)], rbuf1[bA], sem_g1[bA])

        def outer_body(outer, carry):
            for k in range(_NSLOT):
                step(outer, k)
            return carry

        lax.fori_loop(0, n_outer, outer_body, 0)

        for j in range(_NSLOT):
            c = n_full - _NSLOT + j
            pltpu.make_async_copy(
                wbuf[c % _NSLOT], out_slice(c), sem_w[c % _NSLOT]).wait()

        if tail:
            off = n_full * _CHUNK
            ti0 = i0_all.at[pl.ds(off, tail)]
            ti1 = i1_all.at[pl.ds(off, tail)]
            tb0 = rbuf0[0].at[pl.ds(0, tail)]
            tb1 = rbuf1[0].at[pl.ds(0, tail)]
            cp0 = pltpu.async_copy(y0_hbm.at[ti0], tb0, sem_g0[0])
            cp1 = pltpu.async_copy(y1_hbm.at[ti1], tb1, sem_g1[0])
            cp0.wait()
            cp1.wait()
            expand_add(rbuf0[0], rbuf1[0], wbuf[0], tail)
            pltpu.sync_copy(wbuf[0].at[pl.ds(0, tail)],
                            out_hbm.at[pl.ds(base + off, tail)])

    return gather_add


def _col_perm(d):
    perm = np.empty(d, dtype=np.int32)
    for j in range(d // 2):
        perm[j] = 32 * (j // 16) + j % 16
        perm[d // 2 + j] = 32 * (j // 16) + 16 + j % 16
    return perm


def kernel(x0, x1, src0, src1, W0, b0, W1, b1):
    D = x0.shape[1]
    perm = _col_perm(D)
    y0, y1 = _node_transform(
        x0, x1, W0[:, perm], b0[perm], W1[:, perm], b1[perm])
    E = src0.shape[0]
    return _make_gather_add(E, D)(y0, y1, src0, src1)


# speedup vs baseline: 1.0003x; 1.0003x over previous
import functools

import jax
import jax.numpy as jnp
import numpy as np
from jax import lax
from jax.experimental import pallas as pl
from jax.experimental.pallas import tpu as pltpu
from jax.experimental.pallas import tpu_sc as plsc


def _pack(y):
    d = y.shape[1]
    u = lax.bitcast_convert_type(
        y.astype(jnp.bfloat16), jnp.uint16).astype(jnp.int32)
    return u[:, :d // 2] | (u[:, d // 2:] << 16)


def _node_transform_body(x0_ref, x1_ref, w0_ref, w1_ref, b0_ref, b1_ref,
                         y0_ref, y1_ref):
    y0_ref[...] = _pack(
        jnp.dot(x0_ref[...], w0_ref[...], preferred_element_type=jnp.float32)
        + b0_ref[...])
    y1_ref[...] = _pack(
        jnp.dot(x1_ref[...], w1_ref[...], preferred_element_type=jnp.float32)
        + b1_ref[...])


def _node_transform(x0, x1, W0, b0, W1, b1):
    n, d = x0.shape
    grid = 10
    rows = n // grid
    row_spec = pl.BlockSpec((rows, d), lambda i: (i, 0))
    full_spec = pl.BlockSpec((d, d), lambda i: (0, 0))
    bias_spec = pl.BlockSpec((1, d), lambda i: (0, 0))
    packed_spec = pl.BlockSpec((rows, d // 2), lambda i: (i, 0))
    return pl.pallas_call(
        _node_transform_body,
        grid=(grid,),
        in_specs=[row_spec, row_spec, full_spec, full_spec, bias_spec,
                  bias_spec],
        out_specs=[packed_spec, packed_spec],
        out_shape=[
            jax.ShapeDtypeStruct((n, d // 2), jnp.int32),
            jax.ShapeDtypeStruct((n, d // 2), jnp.int32),
        ],
    )(x0, x1, W0, W1, b0.reshape(1, d), b1.reshape(1, d))


_CHUNK = 128
_NSLOT = 3


def _make_gather_add(E, D):
    info = plsc.get_sparse_core_info()
    nw = info.num_cores * info.num_subcores
    e_per_w = E // nw
    assert e_per_w * nw == E and e_per_w % 8 == 0
    n_full = e_per_w // _CHUNK
    tail = e_per_w - n_full * _CHUNK
    assert tail % 8 == 0
    dw = D // 2
    n_steps = n_full + 2
    n_outer = (n_steps + _NSLOT - 1) // _NSLOT
    mesh = plsc.VectorSubcoreMesh(core_axis_name="c", subcore_axis_name="s")

    @functools.partial(
        pl.kernel,
        mesh=mesh,
        out_type=jax.ShapeDtypeStruct((E, D), jnp.float32),
        compiler_params=pltpu.CompilerParams(use_tc_tiling_on_sc=False),
        scratch_types=(
            [pltpu.VMEM((e_per_w,), jnp.int32)] * 2
            + [pltpu.VMEM((_CHUNK, dw), jnp.int32)] * (2 * _NSLOT)
            + [pltpu.VMEM((_CHUNK, D), jnp.float32)] * _NSLOT
            + [pltpu.SemaphoreType.DMA] * (3 * _NSLOT)
        ),
    )
    def gather_add(y0_hbm, y1_hbm, src0_hbm, src1_hbm, out_hbm,
                   i0_all, i1_all, *bufs_and_sems):
        rbuf0 = bufs_and_sems[:_NSLOT]
        rbuf1 = bufs_and_sems[_NSLOT:2 * _NSLOT]
        wbuf = bufs_and_sems[2 * _NSLOT:3 * _NSLOT]
        sem_g0 = bufs_and_sems[3 * _NSLOT:4 * _NSLOT]
        sem_g1 = bufs_and_sems[4 * _NSLOT:5 * _NSLOT]
        sem_w = bufs_and_sems[5 * _NSLOT:6 * _NSLOT]
        wid = lax.axis_index("s") * info.num_cores + lax.axis_index("c")
        base = wid * e_per_w

        pltpu.sync_copy(src0_hbm.at[pl.ds(base, e_per_w)], i0_all)
        pltpu.sync_copy(src1_hbm.at[pl.ds(base, e_per_w)], i1_all)

        def idx0(c):
            return i0_all.at[pl.ds(c * _CHUNK, _CHUNK)]

        def idx1(c):
            return i1_all.at[pl.ds(c * _CHUNK, _CHUNK)]

        def out_slice(c):
            return out_hbm.at[pl.ds(base + c * _CHUNK, _CHUNK)]

        def expand_add(rb0, rb1, wb, count):
            mask = jnp.int32(-65536)

            @plsc.parallel_loop(0, count, unroll=8)
            def _(r):
                for q in range(D // 32):
                    w0 = rb0[r, pl.ds(16 * q, 16)]
                    w1 = rb1[r, pl.ds(16 * q, 16)]
                    lo = (lax.bitcast_convert_type(w0 << 16, jnp.float32)
                          + lax.bitcast_convert_type(w1 << 16, jnp.float32))
                    hi = (lax.bitcast_convert_type(w0 & mask, jnp.float32)
                          + lax.bitcast_convert_type(w1 & mask, jnp.float32))
                    wb[r, pl.ds(32 * q, 16)] = lo
                    wb[r, pl.ds(32 * q + 16, 16)] = hi

        def step(outer, k):
            c2 = outer * _NSLOT + k
            cA = c2
            cC = c2 - 2
            bA = k
            bC = (k - 2) % _NSLOT

            @pl.when(jnp.logical_and(cC >= 0, cC < n_full))
            def _():
                pltpu.make_async_copy(
                    y0_hbm.at[idx0(cC)], rbuf0[bC], sem_g0[bC]).wait()
                pltpu.make_async_copy(
                    y1_hbm.at[idx1(cC)], rbuf1[bC], sem_g1[bC]).wait()

                @pl.when(cC >= _NSLOT)
                def _():
                    pltpu.make_async_copy(
                        wbuf[bC], out_slice(cC - _NSLOT), sem_w[bC]).wait()

                expand_add(rbuf0[bC], rbuf1[bC], wbuf[bC], _CHUNK)
                pltpu.async_copy(wbuf[bC], out_slice(cC), sem_w[bC])

            @pl.when(cA < n_full)
            def _():
                pltpu.async_copy(y0_hbm.at[idx0(cA)], rbuf0[bA], sem_g0[bA])
                pltpu.async_copy(y1_hbm.at[idx1(cA)], rbuf1[bA], sem_g1[bA])

        def outer_body(outer, carry):
            for k in range(_NSLOT):
                step(outer, k)
            return carry

        lax.fori_loop(0, n_outer, outer_body, 0)

        for j in range(_NSLOT):
            c = n_full - _NSLOT + j
            pltpu.make_async_copy(
                wbuf[c % _NSLOT], out_slice(c), sem_w[c % _NSLOT]).wait()

        if tail:
            off = n_full * _CHUNK
            ti0 = i0_all.at[pl.ds(off, tail)]
            ti1 = i1_all.at[pl.ds(off, tail)]
            tb0 = rbuf0[0].at[pl.ds(0, tail)]
            tb1 = rbuf1[0].at[pl.ds(0, tail)]
            cp0 = pltpu.async_copy(y0_hbm.at[ti0], tb0, sem_g0[0])
            cp1 = pltpu.async_copy(y1_hbm.at[ti1], tb1, sem_g1[0])
            cp0.wait()
            cp1.wait()
            expand_add(rbuf0[0], rbuf1[0], wbuf[0], tail)
            pltpu.sync_copy(wbuf[0].at[pl.ds(0, tail)],
                            out_hbm.at[pl.ds(base + off, tail)])

    return gather_add


def _col_perm(d):
    perm = np.empty(d, dtype=np.int32)
    for j in range(d // 2):
        perm[j] = 32 * (j // 16) + j % 16
        perm[d // 2 + j] = 32 * (j // 16) + 16 + j % 16
    return perm


def kernel(x0, x1, src0, src1, W0, b0, W1, b1):
    D = x0.shape[1]
    perm = _col_perm(D)
    y0, y1 = _node_transform(
        x0, x1, W0[:, perm], b0[perm], W1[:, perm], b1[perm])
    E = src0.shape[0]
    return _make_gather_add(E, D)(y0, y1, src0, src1)


# speedup vs baseline: 1.0526x; 1.0522x over previous
import functools

import jax
import jax.numpy as jnp
import numpy as np
from jax import lax
from jax.experimental import pallas as pl
from jax.experimental.pallas import tpu as pltpu
from jax.experimental.pallas import tpu_sc as plsc


def _pack(y):
    d = y.shape[1]
    u = lax.bitcast_convert_type(
        y.astype(jnp.bfloat16), jnp.uint16).astype(jnp.int32)
    return u[:, :d // 2] | (u[:, d // 2:] << 16)


def _node_transform_body(x0_ref, x1_ref, w0_ref, w1_ref, b0_ref, b1_ref,
                         y0_ref, y1_ref):
    y0_ref[...] = _pack(
        jnp.dot(x0_ref[...], w0_ref[...], preferred_element_type=jnp.float32)
        + b0_ref[...])
    y1_ref[...] = _pack(
        jnp.dot(x1_ref[...], w1_ref[...], preferred_element_type=jnp.float32)
        + b1_ref[...])


def _node_transform(x0, x1, W0, b0, W1, b1):
    n, d = x0.shape
    grid = 2
    rows = n // grid
    row_spec = pl.BlockSpec((rows, d), lambda i: (i, 0))
    full_spec = pl.BlockSpec((d, d), lambda i: (0, 0))
    bias_spec = pl.BlockSpec((1, d), lambda i: (0, 0))
    packed_spec = pl.BlockSpec((rows, d // 2), lambda i: (i, 0))
    return pl.pallas_call(
        _node_transform_body,
        grid=(grid,),
        in_specs=[row_spec, row_spec, full_spec, full_spec, bias_spec,
                  bias_spec],
        out_specs=[packed_spec, packed_spec],
        out_shape=[
            jax.ShapeDtypeStruct((n, d // 2), jnp.int32),
            jax.ShapeDtypeStruct((n, d // 2), jnp.int32),
        ],
    )(x0, x1, W0, W1, b0.reshape(1, d), b1.reshape(1, d))


_CHUNK = 128
_NSLOT = 3


def _make_gather_add(E, D):
    info = plsc.get_sparse_core_info()
    nw = info.num_cores * info.num_subcores
    e_per_w = E // nw
    assert e_per_w * nw == E and e_per_w % 8 == 0
    n_full = e_per_w // _CHUNK
    tail = e_per_w - n_full * _CHUNK
    assert tail % 8 == 0
    dw = D // 2
    n_steps = n_full + 2
    n_outer = (n_steps + _NSLOT - 1) // _NSLOT
    mesh = plsc.VectorSubcoreMesh(core_axis_name="c", subcore_axis_name="s")

    @functools.partial(
        pl.kernel,
        mesh=mesh,
        out_type=jax.ShapeDtypeStruct((E, D), jnp.float32),
        compiler_params=pltpu.CompilerParams(use_tc_tiling_on_sc=False),
        scratch_types=(
            [pltpu.VMEM((e_per_w,), jnp.int32)] * 2
            + [pltpu.VMEM((_CHUNK, dw), jnp.int32)] * (2 * _NSLOT)
            + [pltpu.VMEM((_CHUNK, D), jnp.float32)] * _NSLOT
            + [pltpu.SemaphoreType.DMA] * (3 * _NSLOT)
        ),
    )
    def gather_add(y0_hbm, y1_hbm, src0_hbm, src1_hbm, out_hbm,
                   i0_all, i1_all, *bufs_and_sems):
        rbuf0 = bufs_and_sems[:_NSLOT]
        rbuf1 = bufs_and_sems[_NSLOT:2 * _NSLOT]
        wbuf = bufs_and_sems[2 * _NSLOT:3 * _NSLOT]
        sem_g0 = bufs_and_sems[3 * _NSLOT:4 * _NSLOT]
        sem_g1 = bufs_and_sems[4 * _NSLOT:5 * _NSLOT]
        sem_w = bufs_and_sems[5 * _NSLOT:6 * _NSLOT]
        wid = lax.axis_index("s") * info.num_cores + lax.axis_index("c")
        base = wid * e_per_w

        pltpu.sync_copy(src0_hbm.at[pl.ds(base, e_per_w)], i0_all)
        pltpu.sync_copy(src1_hbm.at[pl.ds(base, e_per_w)], i1_all)

        def idx0(c):
            return i0_all.at[pl.ds(c * _CHUNK, _CHUNK)]

        def idx1(c):
            return i1_all.at[pl.ds(c * _CHUNK, _CHUNK)]

        def out_slice(c):
            return out_hbm.at[pl.ds(base + c * _CHUNK, _CHUNK)]

        def expand_add(rb0, rb1, wb, count):
            mask = jnp.int32(-65536)

            @plsc.parallel_loop(0, count, unroll=8)
            def _(r):
                for q in range(D // 32):
                    w0 = rb0[r, pl.ds(16 * q, 16)]
                    w1 = rb1[r, pl.ds(16 * q, 16)]
                    lo = (lax.bitcast_convert_type(w0 << 16, jnp.float32)
                          + lax.bitcast_convert_type(w1 << 16, jnp.float32))
                    hi = (lax.bitcast_convert_type(w0 & mask, jnp.float32)
                          + lax.bitcast_convert_type(w1 & mask, jnp.float32))
                    wb[r, pl.ds(32 * q, 16)] = lo
                    wb[r, pl.ds(32 * q + 16, 16)] = hi

        def step(outer, k):
            c2 = outer * _NSLOT + k
            cA = c2
            cC = c2 - 2
            bA = k
            bC = (k - 2) % _NSLOT

            @pl.when(jnp.logical_and(cC >= 0, cC < n_full))
            def _():
                pltpu.make_async_copy(
                    y0_hbm.at[idx0(cC)], rbuf0[bC], sem_g0[bC]).wait()
                pltpu.make_async_copy(
                    y1_hbm.at[idx1(cC)], rbuf1[bC], sem_g1[bC]).wait()

                @pl.when(cC >= _NSLOT)
                def _():
                    pltpu.make_async_copy(
                        wbuf[bC], out_slice(cC - _NSLOT), sem_w[bC]).wait()

                expand_add(rbuf0[bC], rbuf1[bC], wbuf[bC], _CHUNK)
                pltpu.async_copy(wbuf[bC], out_slice(cC), sem_w[bC])

            @pl.when(cA < n_full)
            def _():
                pltpu.async_copy(y0_hbm.at[idx0(cA)], rbuf0[bA], sem_g0[bA])
                pltpu.async_copy(y1_hbm.at[idx1(cA)], rbuf1[bA], sem_g1[bA])

        def outer_body(outer, carry):
            for k in range(_NSLOT):
                step(outer, k)
            return carry

        lax.fori_loop(0, n_outer, outer_body, 0)

        for j in range(_NSLOT):
            c = n_full - _NSLOT + j
            pltpu.make_async_copy(
                wbuf[c % _NSLOT], out_slice(c), sem_w[c % _NSLOT]).wait()

        if tail:
            off = n_full * _CHUNK
            ti0 = i0_all.at[pl.ds(off, tail)]
            ti1 = i1_all.at[pl.ds(off, tail)]
            tb0 = rbuf0[0].at[pl.ds(0, tail)]
            tb1 = rbuf1[0].at[pl.ds(0, tail)]
            cp0 = pltpu.async_copy(y0_hbm.at[ti0], tb0, sem_g0[0])
            cp1 = pltpu.async_copy(y1_hbm.at[ti1], tb1, sem_g1[0])
            cp0.wait()
            cp1.wait()
            expand_add(rbuf0[0], rbuf1[0], wbuf[0], tail)
            pltpu.sync_copy(wbuf[0].at[pl.ds(0, tail)],
                            out_hbm.at[pl.ds(base + off, tail)])

    return gather_add


def _col_perm(d):
    perm = np.empty(d, dtype=np.int32)
    for j in range(d // 2):
        perm[j] = 32 * (j // 16) + j % 16
        perm[d // 2 + j] = 32 * (j // 16) + 16 + j % 16
    return perm


def kernel(x0, x1, src0, src1, W0, b0, W1, b1):
    D = x0.shape[1]
    perm = _col_perm(D)
    y0, y1 = _node_transform(
        x0, x1, W0[:, perm], b0[perm], W1[:, perm], b1[perm])
    E = src0.shape[0]
    return _make_gather_add(E, D)(y0, y1, src0, src1)
